# baseline (speedup 1.0000x reference)
import functools

import jax
import jax.numpy as jnp
from jax import lax
from jax.experimental import pallas as pl
from jax.experimental.pallas import tpu as pltpu
from jax.experimental.pallas import tpu_sc as plsc

N = 10000
E = 320000
D_IN = 128
D_OUT = 10

NUM_CORES = 2
NUM_SUBCORES = 16
NW = NUM_CORES * NUM_SUBCORES
LANES = 16

BLK = 128
BPW = (E // BLK) // NW
W_MAIN = BPW * BLK
W_REM = E - W_MAIN * NW
REM_BASE = W_MAIN * NW
VECS = W_MAIN // LANES
RVECS = W_REM // LANES
NVECS = N // LANES
UNROLL = 48

_mesh = plsc.VectorSubcoreMesh(
    core_axis_name="c", subcore_axis_name="s",
    num_cores=NUM_CORES, num_subcores=NUM_SUBCORES)

_sc_params = pltpu.CompilerParams(needs_layout_passes=False)


@functools.partial(
    pl.kernel,
    out_type=jax.ShapeDtypeStruct((NW, N), jnp.float32),
    mesh=_mesh,
    compiler_params=_sc_params,
    scratch_types=[
        pltpu.VMEM((2, W_MAIN), jnp.int32),
        pltpu.VMEM((2, W_REM), jnp.int32),
        pltpu.VMEM((N,), jnp.float32),
        pltpu.SemaphoreType.DMA,
        pltpu.SemaphoreType.DMA,
    ],
)
def _count_kernel(ei_hbm, out_hbm, ei_v, rem_v, acc_v, sem1, sem2):
    wid = lax.axis_index("s") * NUM_CORES + lax.axis_index("c")
    base = wid * W_MAIN
    c1 = pltpu.async_copy(ei_hbm.at[:, pl.ds(base, W_MAIN)], ei_v, sem1)
    c2 = pltpu.async_copy(ei_hbm.at[:, pl.ds(REM_BASE, W_REM)], rem_v, sem2)

    zeros = jnp.zeros((LANES,), jnp.float32)

    @plsc.parallel_loop(0, NVECS, 1, unroll=UNROLL)
    def _(i):
        acc_v[pl.ds(i * LANES, LANES)] = zeros

    c1.wait()
    c2.wait()

    ones = jnp.ones((LANES,), jnp.float32)

    @plsc.parallel_loop(0, VECS, 1, unroll=UNROLL)
    def _(i):
        idx = ei_v[0, pl.ds(i * LANES, LANES)]
        plsc.addupdate_scatter(acc_v, [idx], ones)

    @pl.when(wid == NW - 1)
    def _():
        @plsc.parallel_loop(0, RVECS, 1, unroll=UNROLL)
        def _(i):
            idx = rem_v[0, pl.ds(i * LANES, LANES)]
            plsc.addupdate_scatter(acc_v, [idx], ones)

    pltpu.sync_copy(acc_v, out_hbm.at[wid])


@functools.partial(
    pl.kernel,
    out_type=jax.ShapeDtypeStruct((NW, N), jnp.float32),
    mesh=_mesh,
    compiler_params=_sc_params,
    scratch_types=[
        pltpu.VMEM((2, W_MAIN), jnp.int32),
        pltpu.VMEM((2, W_REM), jnp.int32),
        pltpu.VMEM((N,), jnp.float32),
        pltpu.VMEM((N,), jnp.float32),
        pltpu.SemaphoreType.DMA,
        pltpu.SemaphoreType.DMA,
        pltpu.SemaphoreType.DMA,
    ],
)
def _edge_kernel(ei_hbm, dinv_hbm, out_hbm, ei_v, rem_v, dinv_v, acc_v,
                 sem1, sem2, sem3):
    wid = lax.axis_index("s") * NUM_CORES + lax.axis_index("c")
    base = wid * W_MAIN
    c1 = pltpu.async_copy(ei_hbm.at[:, pl.ds(base, W_MAIN)], ei_v, sem1)
    c2 = pltpu.async_copy(ei_hbm.at[:, pl.ds(REM_BASE, W_REM)], rem_v, sem2)
    c3 = pltpu.async_copy(dinv_hbm, dinv_v, sem3)

    zeros = jnp.zeros((LANES,), jnp.float32)

    @plsc.parallel_loop(0, NVECS, 1, unroll=UNROLL)
    def _(i):
        acc_v[pl.ds(i * LANES, LANES)] = zeros

    c1.wait()
    c2.wait()
    c3.wait()

    @plsc.parallel_loop(0, VECS, 1, unroll=UNROLL)
    def _(i):
        sl = pl.ds(i * LANES, LANES)
        src = ei_v[0, sl]
        dst = ei_v[1, sl]
        vals = plsc.load_gather(dinv_v, [src])
        plsc.addupdate_scatter(acc_v, [dst], vals)

    @pl.when(wid == NW - 1)
    def _():
        @plsc.parallel_loop(0, RVECS, 1, unroll=UNROLL)
        def _(i):
            sl = pl.ds(i * LANES, LANES)
            src = rem_v[0, sl]
            dst = rem_v[1, sl]
            vals = plsc.load_gather(dinv_v, [src])
            plsc.addupdate_scatter(acc_v, [dst], vals)

    pltpu.sync_copy(acc_v, out_hbm.at[wid])


def _deg_body(partial_ref, dinv_ref, winv_ref):
    deg = jnp.sum(partial_ref[...], axis=0) + 1.0
    dinv_ref[...] = lax.rsqrt(deg)
    winv_ref[...] = 1.0 / deg


_deg_kernel = pl.pallas_call(
    _deg_body,
    out_shape=(
        jax.ShapeDtypeStruct((N,), jnp.float32),
        jax.ShapeDtypeStruct((N,), jnp.float32),
    ),
)


def _final_body(sp_ref, dinv_ref, winv_ref, x_ref, wg_ref, bg_ref, wo_ref, bo_ref, out_ref):
    s = jnp.sum(sp_ref[...], axis=0)
    w = (dinv_ref[...] * s + winv_ref[...]).reshape(1, N)
    v = jnp.dot(w, x_ref[...], preferred_element_type=jnp.float32)
    h = lax.dot_general(v, wg_ref[...], (((1,), (1,)), ((), ())),
                        preferred_element_type=jnp.float32)
    h = h * (1.0 / N) + bg_ref[...]
    logits = lax.dot_general(h, wo_ref[...], (((1,), (1,)), ((), ())),
                             preferred_element_type=jnp.float32)
    logits = logits + bo_ref[...]
    m = jnp.max(logits, axis=1, keepdims=True)
    y = logits - m
    out_ref[...] = y - jnp.log(jnp.sum(jnp.exp(y), axis=1, keepdims=True))


_final_kernel = pl.pallas_call(
    _final_body,
    out_shape=jax.ShapeDtypeStruct((1, D_OUT), jnp.float32),
)


def kernel(x, ei, W_gcn, b_gcn, W_out, b_out):
    partial_cnt = _count_kernel(ei)
    dinv, winv = _deg_kernel(partial_cnt)
    partial_s = _edge_kernel(ei, dinv)
    out = _final_kernel(
        partial_s, dinv, winv, x,
        W_gcn, b_gcn.reshape(1, D_IN), W_out, b_out.reshape(1, D_OUT))
    return out.reshape(D_OUT)


# speedup vs baseline: 1.0403x; 1.0403x over previous
import functools

import jax
import jax.numpy as jnp
from jax import lax
from jax.experimental import pallas as pl
from jax.experimental.pallas import tpu as pltpu
from jax.experimental.pallas import tpu_sc as plsc

N = 10000
E = 320000
D_IN = 128
D_OUT = 10

NUM_CORES = 2
NUM_SUBCORES = 16
NW = NUM_CORES * NUM_SUBCORES
LANES = 16

BLK = 128
BPW = (E // BLK) // NW
W_MAIN = BPW * BLK
W_REM = E - W_MAIN * NW
REM_BASE = W_MAIN * NW
VECS = W_MAIN // LANES
RVECS = W_REM // LANES
NVECS = N // LANES
UNROLL = 8

_mesh = plsc.VectorSubcoreMesh(
    core_axis_name="c", subcore_axis_name="s",
    num_cores=NUM_CORES, num_subcores=NUM_SUBCORES)

_sc_params = pltpu.CompilerParams(needs_layout_passes=False)


@functools.partial(
    pl.kernel,
    out_type=jax.ShapeDtypeStruct((NW, N), jnp.float32),
    mesh=_mesh,
    compiler_params=_sc_params,
    scratch_types=[
        pltpu.VMEM((2, W_MAIN), jnp.int32),
        pltpu.VMEM((2, W_REM), jnp.int32),
        pltpu.VMEM((N,), jnp.float32),
        pltpu.SemaphoreType.DMA,
        pltpu.SemaphoreType.DMA,
    ],
)
def _count_kernel(ei_hbm, out_hbm, ei_v, rem_v, acc_v, sem1, sem2):
    wid = lax.axis_index("s") * NUM_CORES + lax.axis_index("c")
    base = wid * W_MAIN
    c1 = pltpu.async_copy(ei_hbm.at[:, pl.ds(base, W_MAIN)], ei_v, sem1)
    c2 = pltpu.async_copy(ei_hbm.at[:, pl.ds(REM_BASE, W_REM)], rem_v, sem2)

    zeros = jnp.zeros((LANES,), jnp.float32)

    @plsc.parallel_loop(0, NVECS, 1, unroll=UNROLL)
    def _(i):
        acc_v[pl.ds(i * LANES, LANES)] = zeros

    c1.wait()
    c2.wait()

    ones = jnp.ones((LANES,), jnp.float32)

    @plsc.parallel_loop(0, VECS, 1, unroll=UNROLL)
    def _(i):
        idx = ei_v[0, pl.ds(i * LANES, LANES)]
        plsc.addupdate_scatter(acc_v, [idx], ones)

    @pl.when(wid == NW - 1)
    def _():
        @plsc.parallel_loop(0, RVECS, 1, unroll=UNROLL)
        def _(i):
            idx = rem_v[0, pl.ds(i * LANES, LANES)]
            plsc.addupdate_scatter(acc_v, [idx], ones)

    pltpu.sync_copy(acc_v, out_hbm.at[wid])


@functools.partial(
    pl.kernel,
    out_type=jax.ShapeDtypeStruct((NW, N), jnp.float32),
    mesh=_mesh,
    compiler_params=_sc_params,
    scratch_types=[
        pltpu.VMEM((2, W_MAIN), jnp.int32),
        pltpu.VMEM((2, W_REM), jnp.int32),
        pltpu.VMEM((N,), jnp.float32),
        pltpu.VMEM((N,), jnp.float32),
        pltpu.SemaphoreType.DMA,
        pltpu.SemaphoreType.DMA,
        pltpu.SemaphoreType.DMA,
    ],
)
def _edge_kernel(ei_hbm, dinv_hbm, out_hbm, ei_v, rem_v, dinv_v, acc_v,
                 sem1, sem2, sem3):
    wid = lax.axis_index("s") * NUM_CORES + lax.axis_index("c")
    base = wid * W_MAIN
    c1 = pltpu.async_copy(ei_hbm.at[:, pl.ds(base, W_MAIN)], ei_v, sem1)
    c2 = pltpu.async_copy(ei_hbm.at[:, pl.ds(REM_BASE, W_REM)], rem_v, sem2)
    c3 = pltpu.async_copy(dinv_hbm, dinv_v, sem3)

    zeros = jnp.zeros((LANES,), jnp.float32)

    @plsc.parallel_loop(0, NVECS, 1, unroll=UNROLL)
    def _(i):
        acc_v[pl.ds(i * LANES, LANES)] = zeros

    c1.wait()
    c2.wait()
    c3.wait()

    @plsc.parallel_loop(0, VECS, 1, unroll=UNROLL)
    def _(i):
        sl = pl.ds(i * LANES, LANES)
        src = ei_v[0, sl]
        dst = ei_v[1, sl]
        vals = plsc.load_gather(dinv_v, [src])
        plsc.addupdate_scatter(acc_v, [dst], vals)

    @pl.when(wid == NW - 1)
    def _():
        @plsc.parallel_loop(0, RVECS, 1, unroll=UNROLL)
        def _(i):
            sl = pl.ds(i * LANES, LANES)
            src = rem_v[0, sl]
            dst = rem_v[1, sl]
            vals = plsc.load_gather(dinv_v, [src])
            plsc.addupdate_scatter(acc_v, [dst], vals)

    pltpu.sync_copy(acc_v, out_hbm.at[wid])


def _deg_body(partial_ref, dinv_ref, winv_ref):
    deg = jnp.sum(partial_ref[...], axis=0) + 1.0
    dinv_ref[...] = lax.rsqrt(deg)
    winv_ref[...] = 1.0 / deg


_deg_kernel = pl.pallas_call(
    _deg_body,
    out_shape=(
        jax.ShapeDtypeStruct((N,), jnp.float32),
        jax.ShapeDtypeStruct((N,), jnp.float32),
    ),
)


def _final_body(sp_ref, dinv_ref, winv_ref, x_ref, wg_ref, bg_ref, wo_ref, bo_ref, out_ref):
    s = jnp.sum(sp_ref[...], axis=0)
    w = (dinv_ref[...] * s + winv_ref[...]).reshape(1, N)
    v = jnp.dot(w, x_ref[...], preferred_element_type=jnp.float32)
    h = lax.dot_general(v, wg_ref[...], (((1,), (1,)), ((), ())),
                        preferred_element_type=jnp.float32)
    h = h * (1.0 / N) + bg_ref[...]
    logits = lax.dot_general(h, wo_ref[...], (((1,), (1,)), ((), ())),
                             preferred_element_type=jnp.float32)
    logits = logits + bo_ref[...]
    m = jnp.max(logits, axis=1, keepdims=True)
    y = logits - m
    out_ref[...] = y - jnp.log(jnp.sum(jnp.exp(y), axis=1, keepdims=True))


_final_kernel = pl.pallas_call(
    _final_body,
    out_shape=jax.ShapeDtypeStruct((1, D_OUT), jnp.float32),
)


def kernel(x, ei, W_gcn, b_gcn, W_out, b_out):
    partial_cnt = _count_kernel(ei)
    dinv, winv = _deg_kernel(partial_cnt)
    partial_s = _edge_kernel(ei, dinv)
    out = _final_kernel(
        partial_s, dinv, winv, x,
        W_gcn, b_gcn.reshape(1, D_IN), W_out, b_out.reshape(1, D_OUT))
    return out.reshape(D_OUT)


# speedup vs baseline: 1.0416x; 1.0013x over previous
import functools

import jax
import jax.numpy as jnp
from jax import lax
from jax.experimental import pallas as pl
from jax.experimental.pallas import tpu as pltpu
from jax.experimental.pallas import tpu_sc as plsc

N = 10000
E = 320000
D_IN = 128
D_OUT = 10

NUM_CORES = 2
NUM_SUBCORES = 16
NW = NUM_CORES * NUM_SUBCORES
LANES = 16

BLK = 128
BPW = (E // BLK) // NW
W_MAIN = BPW * BLK
W_REM = E - W_MAIN * NW
REM_BASE = W_MAIN * NW
VECS = W_MAIN // LANES
RVECS = W_REM // LANES
NVECS = N // LANES
UNROLL = 4

_mesh = plsc.VectorSubcoreMesh(
    core_axis_name="c", subcore_axis_name="s",
    num_cores=NUM_CORES, num_subcores=NUM_SUBCORES)

_sc_params = pltpu.CompilerParams(needs_layout_passes=False)


@functools.partial(
    pl.kernel,
    out_type=jax.ShapeDtypeStruct((NW, N), jnp.float32),
    mesh=_mesh,
    compiler_params=_sc_params,
    scratch_types=[
        pltpu.VMEM((2, W_MAIN), jnp.int32),
        pltpu.VMEM((2, W_REM), jnp.int32),
        pltpu.VMEM((N,), jnp.float32),
        pltpu.SemaphoreType.DMA,
        pltpu.SemaphoreType.DMA,
    ],
)
def _count_kernel(ei_hbm, out_hbm, ei_v, rem_v, acc_v, sem1, sem2):
    wid = lax.axis_index("s") * NUM_CORES + lax.axis_index("c")
    base = wid * W_MAIN
    c1 = pltpu.async_copy(ei_hbm.at[:, pl.ds(base, W_MAIN)], ei_v, sem1)
    c2 = pltpu.async_copy(ei_hbm.at[:, pl.ds(REM_BASE, W_REM)], rem_v, sem2)

    zeros = jnp.zeros((LANES,), jnp.float32)

    @plsc.parallel_loop(0, NVECS, 1, unroll=UNROLL)
    def _(i):
        acc_v[pl.ds(i * LANES, LANES)] = zeros

    c1.wait()
    c2.wait()

    ones = jnp.ones((LANES,), jnp.float32)

    @plsc.parallel_loop(0, VECS, 1, unroll=UNROLL)
    def _(i):
        idx = ei_v[0, pl.ds(i * LANES, LANES)]
        plsc.addupdate_scatter(acc_v, [idx], ones)

    @pl.when(wid == NW - 1)
    def _():
        @plsc.parallel_loop(0, RVECS, 1, unroll=UNROLL)
        def _(i):
            idx = rem_v[0, pl.ds(i * LANES, LANES)]
            plsc.addupdate_scatter(acc_v, [idx], ones)

    pltpu.sync_copy(acc_v, out_hbm.at[wid])


@functools.partial(
    pl.kernel,
    out_type=jax.ShapeDtypeStruct((NW, N), jnp.float32),
    mesh=_mesh,
    compiler_params=_sc_params,
    scratch_types=[
        pltpu.VMEM((2, W_MAIN), jnp.int32),
        pltpu.VMEM((2, W_REM), jnp.int32),
        pltpu.VMEM((N,), jnp.float32),
        pltpu.VMEM((N,), jnp.float32),
        pltpu.SemaphoreType.DMA,
        pltpu.SemaphoreType.DMA,
        pltpu.SemaphoreType.DMA,
    ],
)
def _edge_kernel(ei_hbm, dinv_hbm, out_hbm, ei_v, rem_v, dinv_v, acc_v,
                 sem1, sem2, sem3):
    wid = lax.axis_index("s") * NUM_CORES + lax.axis_index("c")
    base = wid * W_MAIN
    c1 = pltpu.async_copy(ei_hbm.at[:, pl.ds(base, W_MAIN)], ei_v, sem1)
    c2 = pltpu.async_copy(ei_hbm.at[:, pl.ds(REM_BASE, W_REM)], rem_v, sem2)
    c3 = pltpu.async_copy(dinv_hbm, dinv_v, sem3)

    zeros = jnp.zeros((LANES,), jnp.float32)

    @plsc.parallel_loop(0, NVECS, 1, unroll=UNROLL)
    def _(i):
        acc_v[pl.ds(i * LANES, LANES)] = zeros

    c1.wait()
    c2.wait()
    c3.wait()

    @plsc.parallel_loop(0, VECS, 1, unroll=UNROLL)
    def _(i):
        sl = pl.ds(i * LANES, LANES)
        src = ei_v[0, sl]
        dst = ei_v[1, sl]
        vals = plsc.load_gather(dinv_v, [src])
        plsc.addupdate_scatter(acc_v, [dst], vals)

    @pl.when(wid == NW - 1)
    def _():
        @plsc.parallel_loop(0, RVECS, 1, unroll=UNROLL)
        def _(i):
            sl = pl.ds(i * LANES, LANES)
            src = rem_v[0, sl]
            dst = rem_v[1, sl]
            vals = plsc.load_gather(dinv_v, [src])
            plsc.addupdate_scatter(acc_v, [dst], vals)

    pltpu.sync_copy(acc_v, out_hbm.at[wid])


def _deg_body(partial_ref, dinv_ref, winv_ref):
    deg = jnp.sum(partial_ref[...], axis=0) + 1.0
    dinv_ref[...] = lax.rsqrt(deg)
    winv_ref[...] = 1.0 / deg


_deg_kernel = pl.pallas_call(
    _deg_body,
    out_shape=(
        jax.ShapeDtypeStruct((N,), jnp.float32),
        jax.ShapeDtypeStruct((N,), jnp.float32),
    ),
)


def _final_body(sp_ref, dinv_ref, winv_ref, x_ref, wg_ref, bg_ref, wo_ref, bo_ref, out_ref):
    s = jnp.sum(sp_ref[...], axis=0)
    w = (dinv_ref[...] * s + winv_ref[...]).reshape(1, N)
    v = jnp.dot(w, x_ref[...], preferred_element_type=jnp.float32)
    h = lax.dot_general(v, wg_ref[...], (((1,), (1,)), ((), ())),
                        preferred_element_type=jnp.float32)
    h = h * (1.0 / N) + bg_ref[...]
    logits = lax.dot_general(h, wo_ref[...], (((1,), (1,)), ((), ())),
                             preferred_element_type=jnp.float32)
    logits = logits + bo_ref[...]
    m = jnp.max(logits, axis=1, keepdims=True)
    y = logits - m
    out_ref[...] = y - jnp.log(jnp.sum(jnp.exp(y), axis=1, keepdims=True))


_final_kernel = pl.pallas_call(
    _final_body,
    out_shape=jax.ShapeDtypeStruct((1, D_OUT), jnp.float32),
)


def kernel(x, ei, W_gcn, b_gcn, W_out, b_out):
    partial_cnt = _count_kernel(ei)
    dinv, winv = _deg_kernel(partial_cnt)
    partial_s = _edge_kernel(ei, dinv)
    out = _final_kernel(
        partial_s, dinv, winv, x,
        W_gcn, b_gcn.reshape(1, D_IN), W_out, b_out.reshape(1, D_OUT))
    return out.reshape(D_OUT)
